# baseline (speedup 1.0000x reference)
import functools

import jax
import jax.numpy as jnp
from jax import lax
from jax.experimental import pallas as pl
from jax.experimental.pallas import tpu as pltpu
from jax.experimental.pallas import tpu_sc as plsc

BETA = 0.25
BN = 512
KC = 256
HL = 16


def _cn_kernel(lut_ref, cn_ref):
    lut = lut_ref[...]
    cn_ref[0, :] = jnp.sum(lut * lut, axis=1)


def _argmin_kernel(x_ref, w1_ref, b1_ref, lut_ref, cn_ref, xq_ref, q_ref,
                   *, nkc):
    xq = jnp.dot(x_ref[...], w1_ref[...],
                 preferred_element_type=jnp.float32) + b1_ref[...]
    xq_ref[...] = xq
    rown = jnp.sum(xq * xq, axis=1, keepdims=True)
    xq2 = xq + xq

    bestw = None
    kidxw = None
    for k in range(nkc):
        lut_c = lut_ref[k * KC:(k + 1) * KC, :]
        cn_c = cn_ref[:, k * KC:(k + 1) * KC]
        mm2 = lax.dot_general(xq2, lut_c, (((1,), (1,)), ((), ())),
                              preferred_element_type=jnp.float32)
        d2 = (rown + cn_c) - mm2
        dist = jnp.sqrt(jnp.maximum(d2, 0.0))
        if k == 0:
            bestw = dist
            kidxw = jnp.zeros((BN, KC), jnp.float32)
        else:
            upd = dist < bestw
            kidxw = jnp.where(upd, jnp.float32(k), kidxw)
            bestw = jnp.minimum(bestw, dist)

    m = jnp.min(bestw, axis=1, keepdims=True)
    colf = lax.broadcasted_iota(jnp.int32, (1, KC), 1).astype(jnp.float32)
    gidx = kidxw * jnp.float32(KC) + colf
    idx = jnp.min(jnp.where(bestw == m, gidx, jnp.float32(1e9)),
                  axis=1, keepdims=True)
    q_ref[...] = idx.astype(jnp.int32)


def _post_kernel(xq_ref, xe_ref, w2_ref, b2_ref, q_ref,
                 out_ref, loss_ref, perp_ref, acc_s, hist_s, *, n, cd, kdim):
    i = pl.program_id(0)
    nb = pl.num_programs(0)
    xe = xe_ref[...]
    out_ref[...] = jnp.dot(xe, w2_ref[...],
                           preferred_element_type=jnp.float32) + b2_ref[...]
    diff = xq_ref[...] - xe

    @pl.when(i == 0)
    def _():
        acc_s[0] = 0.0
        hist_s[...] = jnp.zeros((kdim // 128, 128), jnp.float32)

    acc_s[0] += jnp.sum(diff * diff)
    q = q_ref[...]
    ohh = (lax.shift_right_logical(q, 7)
           == lax.broadcasted_iota(jnp.int32, (1, kdim // 128), 1)
           ).astype(jnp.float32)
    ohl = ((q & 127)
           == lax.broadcasted_iota(jnp.int32, (1, 128), 1)
           ).astype(jnp.float32)
    hist_s[...] += lax.dot_general(ohh, ohl, (((0,), (0,)), ((), ())),
                                   preferred_element_type=jnp.float32)

    @pl.when(i == nb - 1)
    def _():
        m = acc_s[0] / jnp.float32(n * cd)
        loss_ref[...] = jnp.reshape((1.0 - BETA) * m + BETA * m, (1, 1))
        p = hist_s[...] / jnp.float32(n)
        perp_ref[...] = jnp.reshape(
            jnp.exp(-jnp.sum(p * jnp.log(p + 1e-10))), (1, 1))


def _make_sc_gather(kdim, cd, n):
    info = plsc.get_sparse_core_info()
    nc = info.num_cores
    nw = nc * info.num_subcores
    bpw = n // nw
    hw = bpw // 2
    mesh = plsc.VectorSubcoreMesh(core_axis_name="c", subcore_axis_name="s")

    @functools.partial(
        pl.kernel, mesh=mesh,
        out_type=jax.ShapeDtypeStruct((n, cd), jnp.float32),
        scratch_types=[
            pltpu.VMEM((2, hw), jnp.int32),
            pltpu.VMEM((hw, cd), jnp.float32),
            pltpu.VMEM((hw, cd), jnp.float32),
            pltpu.SemaphoreType.DMA,
        ],
    )
    def gather_k(table_hbm, idx_hbm, xe_hbm, idx2, rows0, rows1, sem):
        c = lax.axis_index("c")
        s = lax.axis_index("s")
        wid = s * nc + c
        base = wid * bpw
        pltpu.sync_copy(idx_hbm.at[pl.ds(base, hw)], idx2.at[0])
        pltpu.sync_copy(idx_hbm.at[pl.ds(base + hw, hw)], idx2.at[1])
        cp0 = pltpu.async_copy(table_hbm.at[idx2.at[0]], rows0, sem)
        cp1 = pltpu.async_copy(table_hbm.at[idx2.at[1]], rows1, sem)
        cp0.wait()
        cp1.wait()
        pltpu.sync_copy(rows0, xe_hbm.at[pl.ds(base, hw)])
        pltpu.sync_copy(rows1, xe_hbm.at[pl.ds(base + hw, hw)])

    return gather_k


def kernel(x, W1, b1, lookup_table, W2, b2):
    b, d, h, w = x.shape
    kdim, cd = lookup_table.shape
    n = b * h * w
    nb = n // BN
    nkc = kdim // KC

    x_flat = x.transpose(0, 2, 3, 1).reshape(n, d)
    b1r = b1.reshape(1, cd)
    b2r = b2.reshape(1, d)

    cn2 = pl.pallas_call(
        _cn_kernel,
        out_shape=jax.ShapeDtypeStruct((1, kdim), jnp.float32),
    )(lookup_table)

    xq, q3 = pl.pallas_call(
        functools.partial(_argmin_kernel, nkc=nkc),
        grid=(nb,),
        in_specs=[
            pl.BlockSpec((BN, d), lambda i: (i, 0)),
            pl.BlockSpec((d, cd), lambda i: (0, 0)),
            pl.BlockSpec((1, cd), lambda i: (0, 0)),
            pl.BlockSpec((kdim, cd), lambda i: (0, 0)),
            pl.BlockSpec((1, kdim), lambda i: (0, 0)),
        ],
        out_specs=[
            pl.BlockSpec((BN, cd), lambda i: (i, 0)),
            pl.BlockSpec((BN, 1), lambda i: (i, 0)),
        ],
        out_shape=[
            jax.ShapeDtypeStruct((n, cd), jnp.float32),
            jax.ShapeDtypeStruct((n, 1), jnp.int32),
        ],
        compiler_params=pltpu.CompilerParams(
            dimension_semantics=("arbitrary",)),
    )(x_flat, W1, b1r, lookup_table, cn2)

    q = q3.reshape(n)
    x_e = _make_sc_gather(kdim, cd, n)(lookup_table, q)

    out_flat, loss2, perp2 = pl.pallas_call(
        functools.partial(_post_kernel, n=n, cd=cd, kdim=kdim),
        grid=(nb,),
        in_specs=[
            pl.BlockSpec((BN, cd), lambda i: (i, 0)),
            pl.BlockSpec((BN, cd), lambda i: (i, 0)),
            pl.BlockSpec((cd, d), lambda i: (0, 0)),
            pl.BlockSpec((1, d), lambda i: (0, 0)),
            pl.BlockSpec((BN, 1), lambda i: (i, 0)),
        ],
        out_specs=[
            pl.BlockSpec((BN, d), lambda i: (i, 0)),
            pl.BlockSpec((1, 1), lambda i: (0, 0)),
            pl.BlockSpec((1, 1), lambda i: (0, 0)),
        ],
        out_shape=[
            jax.ShapeDtypeStruct((n, d), jnp.float32),
            jax.ShapeDtypeStruct((1, 1), jnp.float32),
            jax.ShapeDtypeStruct((1, 1), jnp.float32),
        ],
        scratch_shapes=[pltpu.SMEM((1,), jnp.float32),
                        pltpu.VMEM((kdim // 128, 128), jnp.float32)],
        compiler_params=pltpu.CompilerParams(
            dimension_semantics=("arbitrary",)),
    )(xq, x_e, W2, b2r, q3)

    out = out_flat.reshape(b, h, w, d).transpose(0, 3, 1, 2)
    q_x = q.reshape(b, h, w)
    return out, loss2[0, 0], q_x, perp2[0, 0]


# speedup vs baseline: 1.1705x; 1.1705x over previous
import functools

import jax
import jax.numpy as jnp
from jax import lax
from jax.experimental import pallas as pl
from jax.experimental.pallas import tpu as pltpu
from jax.experimental.pallas import tpu_sc as plsc

BETA = 0.25
BN = 512
KC = 1024
HL = 16


def _cn_kernel(lut_ref, cn_ref):
    lut = lut_ref[...]
    cn_ref[0, :] = jnp.sum(lut * lut, axis=1)


def _argmin_kernel(x_ref, w1_ref, b1_ref, lut_ref, cn_ref, xq_ref, q_ref,
                   d2_s, *, nkc):
    xq = jnp.dot(x_ref[...], w1_ref[...],
                 preferred_element_type=jnp.float32) + b1_ref[...]
    xq_ref[...] = xq
    rown = jnp.sum(xq * xq, axis=1, keepdims=True)
    xq2 = xq + xq

    m2 = None
    for k in range(nkc):
        lut_c = lut_ref[k * KC:(k + 1) * KC, :]
        cn_c = cn_ref[:, k * KC:(k + 1) * KC]
        mm2 = lax.dot_general(xq2, lut_c, (((1,), (1,)), ((), ())),
                              preferred_element_type=jnp.float32)
        d2 = (rown + cn_c) - mm2
        d2_s[:, k * KC:(k + 1) * KC] = d2
        mt = jnp.min(d2, axis=1, keepdims=True)
        m2 = mt if k == 0 else jnp.minimum(m2, mt)

    m2c = jnp.maximum(m2, 0.0)
    s = jnp.sqrt(m2c)
    c0 = s * s
    cb = lax.bitcast_convert_type(c0, jnp.int32)
    hh = m2c
    for off in (-2, -1, 0, 1, 2, 3, 4):
        cand = lax.bitcast_convert_type(cb + off, jnp.float32)
        eq = jnp.sqrt(jnp.maximum(cand, 0.0)) == s
        hh = jnp.where(eq, cand, hh)

    iota = lax.broadcasted_iota(jnp.int32, (1, nkc * KC), 1).astype(jnp.float32)
    idx = jnp.min(jnp.where(d2_s[...] <= hh, iota, jnp.float32(1e9)),
                  axis=1, keepdims=True)
    q_ref[...] = idx.astype(jnp.int32)


def _post_kernel(xq_ref, xe_ref, w2_ref, b2_ref, q_ref,
                 out_ref, loss_ref, perp_ref, acc_s, hist_s, *, n, cd, kdim):
    i = pl.program_id(0)
    nb = pl.num_programs(0)
    xe = xe_ref[...]
    out_ref[...] = jnp.dot(xe, w2_ref[...],
                           preferred_element_type=jnp.float32) + b2_ref[...]
    diff = xq_ref[...] - xe

    @pl.when(i == 0)
    def _():
        acc_s[0] = 0.0
        hist_s[...] = jnp.zeros((kdim // 128, 128), jnp.float32)

    acc_s[0] += jnp.sum(diff * diff)
    q = q_ref[...]
    ohh = (lax.shift_right_logical(q, 7)
           == lax.broadcasted_iota(jnp.int32, (1, kdim // 128), 1)
           ).astype(jnp.float32)
    ohl = ((q & 127)
           == lax.broadcasted_iota(jnp.int32, (1, 128), 1)
           ).astype(jnp.float32)
    hist_s[...] += lax.dot_general(ohh, ohl, (((0,), (0,)), ((), ())),
                                   preferred_element_type=jnp.float32)

    @pl.when(i == nb - 1)
    def _():
        m = acc_s[0] / jnp.float32(n * cd)
        loss_ref[...] = jnp.reshape((1.0 - BETA) * m + BETA * m, (1, 1))
        p = hist_s[...] / jnp.float32(n)
        perp_ref[...] = jnp.reshape(
            jnp.exp(-jnp.sum(p * jnp.log(p + 1e-10))), (1, 1))


def _make_sc_gather(kdim, cd, n):
    info = plsc.get_sparse_core_info()
    nc = info.num_cores
    nw = nc * info.num_subcores
    bpw = n // nw
    hw = bpw // 2
    mesh = plsc.VectorSubcoreMesh(core_axis_name="c", subcore_axis_name="s")

    @functools.partial(
        pl.kernel, mesh=mesh,
        out_type=jax.ShapeDtypeStruct((n, cd), jnp.float32),
        scratch_types=[
            pltpu.VMEM((2, hw), jnp.int32),
            pltpu.VMEM((hw, cd), jnp.float32),
            pltpu.VMEM((hw, cd), jnp.float32),
            pltpu.SemaphoreType.DMA,
        ],
    )
    def gather_k(table_hbm, idx_hbm, xe_hbm, idx2, rows0, rows1, sem):
        c = lax.axis_index("c")
        s = lax.axis_index("s")
        wid = s * nc + c
        base = wid * bpw
        pltpu.sync_copy(idx_hbm.at[pl.ds(base, hw)], idx2.at[0])
        pltpu.sync_copy(idx_hbm.at[pl.ds(base + hw, hw)], idx2.at[1])
        cp0 = pltpu.async_copy(table_hbm.at[idx2.at[0]], rows0, sem)
        cp1 = pltpu.async_copy(table_hbm.at[idx2.at[1]], rows1, sem)
        cp0.wait()
        cp1.wait()
        pltpu.sync_copy(rows0, xe_hbm.at[pl.ds(base, hw)])
        pltpu.sync_copy(rows1, xe_hbm.at[pl.ds(base + hw, hw)])

    return gather_k


def kernel(x, W1, b1, lookup_table, W2, b2):
    b, d, h, w = x.shape
    kdim, cd = lookup_table.shape
    n = b * h * w
    nb = n // BN
    nkc = kdim // KC

    x_flat = x.transpose(0, 2, 3, 1).reshape(n, d)
    b1r = b1.reshape(1, cd)
    b2r = b2.reshape(1, d)

    cn2 = pl.pallas_call(
        _cn_kernel,
        out_shape=jax.ShapeDtypeStruct((1, kdim), jnp.float32),
    )(lookup_table)

    xq, q3 = pl.pallas_call(
        functools.partial(_argmin_kernel, nkc=nkc),
        grid=(nb,),
        in_specs=[
            pl.BlockSpec((BN, d), lambda i: (i, 0)),
            pl.BlockSpec((d, cd), lambda i: (0, 0)),
            pl.BlockSpec((1, cd), lambda i: (0, 0)),
            pl.BlockSpec((kdim, cd), lambda i: (0, 0)),
            pl.BlockSpec((1, kdim), lambda i: (0, 0)),
        ],
        out_specs=[
            pl.BlockSpec((BN, cd), lambda i: (i, 0)),
            pl.BlockSpec((BN, 1), lambda i: (i, 0)),
        ],
        out_shape=[
            jax.ShapeDtypeStruct((n, cd), jnp.float32),
            jax.ShapeDtypeStruct((n, 1), jnp.int32),
        ],
        scratch_shapes=[pltpu.VMEM((BN, kdim), jnp.float32)],
        compiler_params=pltpu.CompilerParams(
            dimension_semantics=("arbitrary",)),
    )(x_flat, W1, b1r, lookup_table, cn2)

    q = q3.reshape(n)
    x_e = _make_sc_gather(kdim, cd, n)(lookup_table, q)

    out_flat, loss2, perp2 = pl.pallas_call(
        functools.partial(_post_kernel, n=n, cd=cd, kdim=kdim),
        grid=(nb,),
        in_specs=[
            pl.BlockSpec((BN, cd), lambda i: (i, 0)),
            pl.BlockSpec((BN, cd), lambda i: (i, 0)),
            pl.BlockSpec((cd, d), lambda i: (0, 0)),
            pl.BlockSpec((1, d), lambda i: (0, 0)),
            pl.BlockSpec((BN, 1), lambda i: (i, 0)),
        ],
        out_specs=[
            pl.BlockSpec((BN, d), lambda i: (i, 0)),
            pl.BlockSpec((1, 1), lambda i: (0, 0)),
            pl.BlockSpec((1, 1), lambda i: (0, 0)),
        ],
        out_shape=[
            jax.ShapeDtypeStruct((n, d), jnp.float32),
            jax.ShapeDtypeStruct((1, 1), jnp.float32),
            jax.ShapeDtypeStruct((1, 1), jnp.float32),
        ],
        scratch_shapes=[pltpu.SMEM((1,), jnp.float32),
                        pltpu.VMEM((kdim // 128, 128), jnp.float32)],
        compiler_params=pltpu.CompilerParams(
            dimension_semantics=("arbitrary",)),
    )(xq, x_e, W2, b2r, q3)

    out = out_flat.reshape(b, h, w, d).transpose(0, 3, 1, 2)
    q_x = q.reshape(b, h, w)
    return out, loss2[0, 0], q_x, perp2[0, 0]
